# baseline (speedup 1.0000x reference)
import numpy as np
import jax
import jax.numpy as jnp
from jax.experimental import pallas as pl

_HEADS = 4
_HD = 24
_S = 56
_NPIX = _S * _S
_C = 96
_IMGS = 4


def _np_constants():
    u = np.arange(_NPIX) // _S
    v = np.arange(_NPIX) % _S
    tile = (u // 14) * 4 + (v // 14)
    quad = (u // 28) * 2 + (v // 28)
    poolT = (tile[None, :] == np.arange(16)[:, None]).astype(np.float32) / 196.0
    r = np.arange(64)
    h = r % 4
    q = (r // 4) % 4
    m = r // 16
    f1, f2 = q // 2, q % 2
    i, j = m // 2, m % 2
    t_of_r = (2 * f1 + i) * 4 + (2 * f2 + j)
    G = (np.arange(16)[None, :] == t_of_r[:, None]).astype(np.float32)
    E = (np.arange(4)[None, :] == h[:, None]).astype(np.float32)
    ch = np.arange(_C)
    headmask = (ch[None, :] // _HD == h[:, None]).astype(np.float32)
    Hmat = (ch[None, :] // _HD == np.arange(4)[:, None]).astype(np.float32)
    g = np.arange(16)
    qmask = (quad[None, :] == (g // 4)[:, None]).astype(np.float32)
    return poolT, G, E, headmask, Hmat, qmask


def _body(x_ref, fw_ref, fb_ref, vw_ref, vbt_ref, pw_ref, pb_ref, scal_ref,
          poolT_ref, G_ref, E_ref, hm_ref, Hm_ref, qm_ref, out_ref):
    f32 = jnp.float32
    alpha = scal_ref[0, 0]
    beta = scal_ref[1, 0]
    b1 = scal_ref[2, 0]
    b2 = scal_ref[3, 0]
    b3 = scal_ref[4, 0]
    fw = fw_ref[...]
    fb = fb_ref[...]
    vw = vw_ref[...]
    vbt = vbt_ref[...]
    G = G_ref[...]
    hm = hm_ref[...]
    qm = qm_ref[...]
    poolT = poolT_ref[...]
    Hm = Hm_ref[...]
    E = E_ref[...]
    pw = pw_ref[...]
    pb = pb_ref[...]
    one = f32(1.0)
    zero = f32(0.0)
    dn_nt = (((1,), (1,)), ((), ()))
    K = x_ref.shape[0]

    def gt(a, b):
        return jnp.where(a > b, one, zero)

    def ge(a, b):
        return jnp.where(a >= b, one, zero)

    def wsel(rk):
        return jnp.where(rk == 0, one, jnp.where(rk == 1, b1, jnp.where(rk == 2, b2, b3)))

    xxs = [x_ref[k].reshape(_C, _NPIX) for k in range(K)]
    Xpool = [jax.lax.dot_general(xx, poolT, dn_nt, preferred_element_type=f32) for xx in xxs]
    Pf = [jnp.dot(fw, xp, preferred_element_type=f32) + fb for xp in Xpool]
    Pv = [jnp.dot(vw, xp, preferred_element_type=f32) + vbt.T for xp in Xpool]
    C64 = [jax.lax.dot_general(G, p, dn_nt, preferred_element_type=f32) for p in Pf]
    VC64 = [jax.lax.dot_general(G, p, dn_nt, preferred_element_type=f32) for p in Pv]
    Cm = [c * hm for c in C64]
    cn = [c * (jax.lax.rsqrt(jnp.maximum(jnp.sum(c * c, axis=1, keepdims=True), 1e-24)) * alpha)
          for c in Cm]
    Cnfw = [jnp.dot(c, fw, preferred_element_type=f32) for c in cn]
    cb = [jnp.dot(c, fb, preferred_element_type=f32) for c in cn]

    xf = [jnp.dot(fw, xx, preferred_element_type=f32) + fb for xx in xxs]
    sq = [t * t for t in xf]
    n4 = [jnp.dot(Hm, s, preferred_element_type=f32) for s in sq]
    invn = [jax.lax.rsqrt(jnp.maximum(n, 1e-24)) for n in n4]
    inv64 = [jnp.dot(E, iv, preferred_element_type=f32) for iv in invn]
    D = [jnp.dot(c, xx, preferred_element_type=f32) for c, xx in zip(Cnfw, xxs)]
    sim = [jax.nn.sigmoid(beta + (d + c) * iv) for d, c, iv in zip(D, cb, inv64)]

    sim2 = []
    for s in sim:
        S0 = s[0:16]
        S1 = s[16:32]
        S2 = s[32:48]
        S3 = s[48:64]
        r0 = gt(S1, S0) + gt(S2, S0) + gt(S3, S0)
        r1 = ge(S0, S1) + gt(S2, S1) + gt(S3, S1)
        r2 = ge(S0, S2) + ge(S1, S2) + gt(S3, S2)
        r3 = ge(S0, S3) + ge(S1, S3) + ge(S2, S3)
        sim2.append(jnp.concatenate([S0 * (wsel(r0) * qm), S1 * (wsel(r1) * qm),
                                     S2 * (wsel(r2) * qm), S3 * (wsel(r3) * qm)], axis=0))

    rowsum = [jnp.sum(s2, axis=1, keepdims=True) for s2 in sim2]
    aggX = [jax.lax.dot_general(s2, xx, dn_nt, preferred_element_type=f32)
            for s2, xx in zip(sim2, xxs)]
    agg = [jax.lax.dot_general(a, vw, dn_nt, preferred_element_type=f32) + rs * vbt
           for a, rs in zip(aggX, rowsum)]
    OM = [(a + vc) * hm / (rs + 1.0) for a, vc, rs in zip(agg, VC64, rowsum)]
    POM = [jax.lax.dot_general(pw, om, dn_nt, preferred_element_type=f32) for om in OM]
    for k in range(K):
        res = jnp.dot(POM[k], sim2[k], preferred_element_type=f32) + pb
        out_ref[k] = res.reshape(_C, _S, _S)


def kernel(x, f_w, f_b, v_w, v_b, proj_w, proj_b, sim_alpha, sim_beta,
           sim_bis1, sim_bis2, sim_bis3):
    B = x.shape[0]
    consts = [jnp.asarray(a) for a in _np_constants()]
    scal = jnp.stack([sim_alpha[0], sim_beta[0], sim_bis1[0], sim_bis2[0],
                      sim_bis3[0]]).reshape(5, 1)
    fb = f_b.reshape(_C, 1)
    vbt = v_b.reshape(1, _C)
    pb = proj_b.reshape(_C, 1)

    def zspec(s):
        return pl.BlockSpec(s, lambda i: tuple(0 for _ in s))

    out = pl.pallas_call(
        _body,
        grid=(B // _IMGS, ),
        in_specs=[pl.BlockSpec((_IMGS, _C, _S, _S), lambda i: (i, 0, 0, 0)),
                  zspec((_C, _C)), zspec((_C, 1)), zspec((_C, _C)), zspec((1, _C)),
                  zspec((_C, _C)), zspec((_C, 1)), zspec((5, 1)),
                  zspec((16, _NPIX)), zspec((64, 16)), zspec((64, 4)),
                  zspec((64, _C)), zspec((4, _C)), zspec((16, _NPIX))],
        out_specs=pl.BlockSpec((_IMGS, _C, _S, _S), lambda i: (i, 0, 0, 0)),
        out_shape=jax.ShapeDtypeStruct((B, _C, _S, _S), jnp.float32),
    )(x, f_w, fb, v_w, vbt, proj_w, pb, scal, *consts)
    return out


# speedup vs baseline: 1.7319x; 1.7319x over previous
import numpy as np
import jax
import jax.numpy as jnp
from jax.experimental import pallas as pl

_HEADS = 4
_HD = 24
_S = 56
_NPIX = _S * _S
_C = 96
_IMGS = 8


def _np_constants():
    u = np.arange(_NPIX) // _S
    v = np.arange(_NPIX) % _S
    tile = (u // 14) * 4 + (v // 14)
    quad = (u // 28) * 2 + (v // 28)
    poolT = (tile[None, :] == np.arange(16)[:, None]).astype(np.float32) / 196.0
    r = np.arange(64)
    h = r % 4
    q = (r // 4) % 4
    m = r // 16
    f1, f2 = q // 2, q % 2
    i, j = m // 2, m % 2
    t_of_r = (2 * f1 + i) * 4 + (2 * f2 + j)
    G = (np.arange(16)[None, :] == t_of_r[:, None]).astype(np.float32)
    E16 = (np.arange(4)[None, :] == (np.arange(16) % 4)[:, None]).astype(np.float32)
    ch = np.arange(_C)
    headmask = (ch[None, :] // _HD == h[:, None]).astype(np.float32)
    Hmat = (ch[None, :] // _HD == np.arange(4)[:, None]).astype(np.float32)
    g = np.arange(16)
    qmask = (quad[None, :] == (g // 4)[:, None]).astype(np.float32)
    return poolT, G, E16, headmask, Hmat, qmask


def _body(x_ref, fw_ref, fb_ref, vw_ref, vbt_ref, pw_ref, pb_ref, scal_ref,
          poolT_ref, G_ref, E_ref, hm_ref, Hm_ref, qm_ref, out_ref):
    f32 = jnp.float32
    alpha = scal_ref[0, 0]
    beta = scal_ref[1, 0]
    b1 = scal_ref[2, 0]
    b2 = scal_ref[3, 0]
    b3 = scal_ref[4, 0]
    fw = fw_ref[...]
    fb = fb_ref[...]
    vw = vw_ref[...]
    vbt = vbt_ref[...]
    G = G_ref[...]
    hm = hm_ref[...]
    qm = qm_ref[...]
    poolT = poolT_ref[...]
    Hm = Hm_ref[...]
    E = E_ref[...]
    pw = pw_ref[...]
    pb = pb_ref[...]
    one = f32(1.0)
    zero = f32(0.0)
    dn_nt = (((1,), (1,)), ((), ()))
    K = x_ref.shape[0]

    def gt(a, b):
        return jnp.where(a > b, one, zero)

    def ge(a, b):
        return jnp.where(a >= b, one, zero)

    def wsel(rk):
        return jnp.where(rk == 0, one, jnp.where(rk == 1, b1, jnp.where(rk == 2, b2, b3)))

    xxs = [x_ref[k] for k in range(K)]
    Xpool = [jax.lax.dot_general(xx, poolT, dn_nt, preferred_element_type=f32) for xx in xxs]
    Pf = [jnp.dot(fw, xp, preferred_element_type=f32) + fb for xp in Xpool]
    Pv = [jnp.dot(vw, xp, preferred_element_type=f32) + vbt.T for xp in Xpool]
    C64 = [jax.lax.dot_general(G, p, dn_nt, preferred_element_type=f32) for p in Pf]
    VC64 = [jax.lax.dot_general(G, p, dn_nt, preferred_element_type=f32) for p in Pv]
    Cm = [c * hm for c in C64]
    cn = [c * (jax.lax.rsqrt(jnp.maximum(jnp.sum(c * c, axis=1, keepdims=True), 1e-24)) * alpha)
          for c in Cm]
    Cnfw = [jnp.dot(c, fw, preferred_element_type=f32) for c in cn]
    cb = [jnp.dot(c, fb, preferred_element_type=f32) for c in cn]

    xf = [jnp.dot(fw, xx, preferred_element_type=f32) + fb for xx in xxs]
    sq = [t * t for t in xf]
    n4 = [jnp.dot(Hm, s, preferred_element_type=f32) for s in sq]
    invn = [jax.lax.rsqrt(jnp.maximum(n, 1e-24)) * f32(0.5) for n in n4]
    inv16 = [jnp.dot(E, iv, preferred_element_type=f32) for iv in invn]
    D = [jnp.dot(c, xx, preferred_element_type=f32) for c, xx in zip(Cnfw, xxs)]
    hb = beta * f32(0.5)
    half = f32(0.5)

    sim2 = []
    for d, c, iv in zip(D, cb, inv16):
        S0 = jnp.tanh((d[0:16] + c[0:16]) * iv + hb) * half + half
        S1 = jnp.tanh((d[16:32] + c[16:32]) * iv + hb) * half + half
        S2 = jnp.tanh((d[32:48] + c[32:48]) * iv + hb) * half + half
        S3 = jnp.tanh((d[48:64] + c[48:64]) * iv + hb) * half + half
        b01 = gt(S1, S0)
        b02 = gt(S2, S0)
        b03 = gt(S3, S0)
        b12 = gt(S2, S1)
        b13 = gt(S3, S1)
        b23 = gt(S3, S2)
        r0 = b01 + b02 + b03
        r1 = (one - b01) + b12 + b13
        r2 = (f32(2.0) - b02 - b12) + b23
        r3 = f32(6.0) - r0 - r1 - r2
        sim2.append(jnp.concatenate([S0 * (wsel(r0) * qm), S1 * (wsel(r1) * qm),
                                     S2 * (wsel(r2) * qm), S3 * (wsel(r3) * qm)], axis=0))

    rowsum = [jnp.sum(s2, axis=1, keepdims=True) for s2 in sim2]
    aggX = [jax.lax.dot_general(s2, xx, dn_nt, preferred_element_type=f32)
            for s2, xx in zip(sim2, xxs)]
    agg = [jax.lax.dot_general(a, vw, dn_nt, preferred_element_type=f32) + rs * vbt
           for a, rs in zip(aggX, rowsum)]
    OM = [(a + vc) * hm / (rs + 1.0) for a, vc, rs in zip(agg, VC64, rowsum)]
    POM = [jax.lax.dot_general(pw, om, dn_nt, preferred_element_type=f32) for om in OM]
    for k in range(K):
        out_ref[k] = jnp.dot(POM[k], sim2[k], preferred_element_type=f32) + pb


def kernel(x, f_w, f_b, v_w, v_b, proj_w, proj_b, sim_alpha, sim_beta,
           sim_bis1, sim_bis2, sim_bis3):
    B = x.shape[0]
    x3 = x.reshape(B, _C, _NPIX)
    consts = [jnp.asarray(a) for a in _np_constants()]
    scal = jnp.stack([sim_alpha[0], sim_beta[0], sim_bis1[0], sim_bis2[0],
                      sim_bis3[0]]).reshape(5, 1)
    fb = f_b.reshape(_C, 1)
    vbt = v_b.reshape(1, _C)
    pb = proj_b.reshape(_C, 1)

    def zspec(s):
        return pl.BlockSpec(s, lambda i: tuple(0 for _ in s))

    out = pl.pallas_call(
        _body,
        grid=(B // _IMGS, ),
        in_specs=[pl.BlockSpec((_IMGS, _C, _NPIX), lambda i: (i, 0, 0)),
                  zspec((_C, _C)), zspec((_C, 1)), zspec((_C, _C)), zspec((1, _C)),
                  zspec((_C, _C)), zspec((_C, 1)), zspec((5, 1)),
                  zspec((16, _NPIX)), zspec((64, 16)), zspec((16, 4)),
                  zspec((64, _C)), zspec((4, _C)), zspec((16, _NPIX))],
        out_specs=pl.BlockSpec((_IMGS, _C, _NPIX), lambda i: (i, 0, 0)),
        out_shape=jax.ShapeDtypeStruct((B, _C, _NPIX), jnp.float32),
    )(x3, f_w, fb, v_w, vbt, proj_w, pb, scal, *consts)
    return out.reshape(B, _C, _S, _S)
